# baseline (speedup 1.0000x reference)
import functools

import jax
import jax.numpy as jnp
from jax import lax
from jax.experimental import pallas as pl
from jax.experimental.pallas import tpu as pltpu
from jax.experimental.pallas import tpu_sc as plsc

N = 10000
FEAT = 512
NG = 64
NCLS = 10
NEG_INF = float("-inf")

NC = 2
NS = 16
NW = NC * NS
LANE = 16
CH = 64
NCHUNK = 2
TILE_ROWS = CH * NCHUNK
SC_ROWS = NW * TILE_ROWS
NCHW = FEAT // LANE

TCB = 1024
TC_FIRST_BLK = 3
TC_BLOCKS = 7


def _flush(acc, g_cur, regs):
    for c in range(NCHW):
        sl = pl.ds(c * LANE, LANE)
        acc[g_cur, sl] = jnp.maximum(acc[g_cur, sl], regs[c])


def _seg_max_body(x_hbm, ids_hbm, part_hbm, xbuf, ids_v, acc):
    wid = lax.axis_index("c") * NS + lax.axis_index("s")

    def init_g(g, carry):
        for c in range(NCHW):
            acc[g, pl.ds(c * LANE, LANE)] = jnp.full((LANE,), NEG_INF, jnp.float32)
        return carry

    lax.fori_loop(0, NG, init_g, 0)

    base0 = wid * TILE_ROWS
    neg = jnp.full((LANE,), NEG_INF, jnp.float32)
    carry0 = (jnp.int32(0),) + (neg,) * NCHW

    def chunk_body(k, carry):
        base = jnp.minimum(base0 + k * CH, N - CH)
        pltpu.sync_copy(x_hbm.at[pl.ds(base, CH)], xbuf)
        pltpu.sync_copy(ids_hbm.at[pl.ds(base, CH)], ids_v)

        def group_body(rb, carry):
            g_cur = carry[0]
            regs = list(carry[1:])
            idvec = ids_v[pl.ds(rb * LANE, LANE)]
            for j in range(LANE):
                g = idvec[j]
                changed = g != g_cur

                @pl.when(changed)
                def _():
                    _flush(acc, g_cur, regs)

                r = rb * LANE + j
                for c in range(NCHW):
                    row_c = xbuf[r, pl.ds(c * LANE, LANE)]
                    regs[c] = jnp.where(changed, row_c, jnp.maximum(regs[c], row_c))
                g_cur = g
            return (g_cur,) + tuple(regs)

        return lax.fori_loop(0, CH // LANE, group_body, carry)

    carry = lax.fori_loop(0, NCHUNK, chunk_body, carry0)
    _flush(acc, carry[0], list(carry[1:]))
    pltpu.sync_copy(acc, part_hbm.at[wid])


@functools.cache
def _seg_max():
    return functools.partial(
        pl.kernel,
        out_type=jax.ShapeDtypeStruct((NW, NG, FEAT), jnp.float32),
        mesh=plsc.VectorSubcoreMesh(
            core_axis_name="c", subcore_axis_name="s",
            num_cores=NC, num_subcores=NS,
        ),
        scratch_types=[
            pltpu.VMEM((CH, FEAT), jnp.float32),
            pltpu.VMEM((CH,), jnp.int32),
            pltpu.VMEM((NG, FEAT), jnp.float32),
        ],
    )(_seg_max_body)


def _tc_seg_body(sp_ids, x_ref, out_ref):
    i = pl.program_id(0)

    @pl.when(i == 0)
    def _():
        out_ref[...] = jnp.full((NG, FEAT), NEG_INF, jnp.float32)

    base = (TC_FIRST_BLK + i) * TCB
    bend = jnp.minimum(base + TCB, N)
    lo = sp_ids[base]
    hi = sp_ids[bend - 1]
    rows = lax.broadcasted_iota(jnp.int32, (TCB, 1), 0)
    xblk = x_ref[...]

    def lower_bound(val):
        def bb(_, c):
            blo, bhi = c
            mid = (blo + bhi) // 2
            v = sp_ids[jnp.minimum(mid, N - 1)]
            active = blo < bhi
            go = jnp.logical_and(active, v < val)
            stay = jnp.logical_and(active, v >= val)
            return (jnp.where(go, mid + 1, blo), jnp.where(stay, mid, bhi))

        return lax.fori_loop(0, TCB.bit_length(), bb, (base, bend))[0]

    def seg_body(s, rs_local):
        re_local = lower_bound(s + 1) - base
        mask = jnp.logical_and(rows >= rs_local, rows < re_local)
        m = jnp.max(jnp.where(mask, xblk, NEG_INF), axis=0, keepdims=True)
        out_ref[pl.ds(s, 1), :] = jnp.maximum(out_ref[pl.ds(s, 1), :], m)
        return re_local

    lax.fori_loop(lo, hi + 1, seg_body, jnp.int32(0))


def _tc_seg_max(x, batch_index):
    grid_spec = pltpu.PrefetchScalarGridSpec(
        num_scalar_prefetch=1,
        grid=(TC_BLOCKS,),
        in_specs=[
            pl.BlockSpec((TCB, FEAT), lambda i, sp: (TC_FIRST_BLK + i, 0)),
        ],
        out_specs=pl.BlockSpec((NG, FEAT), lambda i, sp: (0, 0)),
    )
    return pl.pallas_call(
        _tc_seg_body,
        grid_spec=grid_spec,
        out_shape=jax.ShapeDtypeStruct((NG, FEAT), jnp.float32),
    )(batch_index, x)


def _finish_body(part_ref, tc_ref, w_ref, b_ref, out_ref, pooled_ref):
    p = tc_ref[...]
    for i in range(NW):
        p = jnp.maximum(p, part_ref[i])
    pooled_ref[...] = p
    out_ref[...] = (
        jnp.dot(p, w_ref[...], preferred_element_type=jnp.float32) + b_ref[...]
    )


def kernel(x, edge_index, batch_index, Wl0, Wr0, a0, b0, Wls, Wrs, atts, bs,
           Wout, bout):
    partials = _seg_max()(x, batch_index)
    tc_part = _tc_seg_max(x, batch_index)
    out, pooled = pl.pallas_call(
        _finish_body,
        out_shape=(
            jax.ShapeDtypeStruct((NG, NCLS), jnp.float32),
            jax.ShapeDtypeStruct((NG, FEAT), jnp.float32),
        ),
    )(partials, tc_part, Wout, bout.reshape(1, NCLS))
    return (out, pooled)


# speedup vs baseline: 1.1753x; 1.1753x over previous
import functools

import jax
import jax.numpy as jnp
from jax import lax
from jax.experimental import pallas as pl
from jax.experimental.pallas import tpu as pltpu
from jax.experimental.pallas import tpu_sc as plsc

N = 10000
FEAT = 512
NG = 64
NCLS = 10
NEG_INF = float("-inf")

NC = 2
NS = 16
NW = NC * NS
LANE = 16
CH = 64
NCHUNK = 2
TILE_ROWS = CH * NCHUNK
SC_ROWS = NW * TILE_ROWS
NCHW = FEAT // LANE

TCB = 512
TC_FIRST_BLK = 7
TC_BLOCKS = 13
TCW = 256


def _flush(acc, g_cur, regs):
    for c in range(NCHW):
        sl = pl.ds(c * LANE, LANE)
        acc[g_cur, sl] = jnp.maximum(acc[g_cur, sl], regs[c])


def _seg_max_body(x_hbm, ids_hbm, part_hbm, xbuf, ids_v, acc):
    wid = lax.axis_index("c") * NS + lax.axis_index("s")

    def init_g(g, carry):
        for c in range(NCHW):
            acc[g, pl.ds(c * LANE, LANE)] = jnp.full((LANE,), NEG_INF, jnp.float32)
        return carry

    lax.fori_loop(0, NG, init_g, 0)

    base0 = wid * TILE_ROWS
    neg = jnp.full((LANE,), NEG_INF, jnp.float32)
    carry0 = (jnp.int32(0),) + (neg,) * NCHW

    def chunk_body(k, carry):
        base = jnp.minimum(base0 + k * CH, N - CH)
        pltpu.sync_copy(x_hbm.at[pl.ds(base, CH)], xbuf)
        pltpu.sync_copy(ids_hbm.at[pl.ds(base, CH)], ids_v)

        def group_body(rb, carry):
            g_cur = carry[0]
            regs = list(carry[1:])
            idvec = ids_v[pl.ds(rb * LANE, LANE)]
            for j in range(LANE):
                g = idvec[j]
                changed = g != g_cur

                @pl.when(changed)
                def _():
                    _flush(acc, g_cur, regs)

                r = rb * LANE + j
                for c in range(NCHW):
                    row_c = xbuf[r, pl.ds(c * LANE, LANE)]
                    regs[c] = jnp.where(changed, row_c, jnp.maximum(regs[c], row_c))
                g_cur = g
            return (g_cur,) + tuple(regs)

        return lax.fori_loop(0, CH // LANE, group_body, carry)

    carry = lax.fori_loop(0, NCHUNK, chunk_body, carry0)
    _flush(acc, carry[0], list(carry[1:]))
    pltpu.sync_copy(acc, part_hbm.at[wid])


@functools.cache
def _seg_max():
    return functools.partial(
        pl.kernel,
        out_type=jax.ShapeDtypeStruct((NW, NG, FEAT), jnp.float32),
        mesh=plsc.VectorSubcoreMesh(
            core_axis_name="c", subcore_axis_name="s",
            num_cores=NC, num_subcores=NS,
        ),
        scratch_types=[
            pltpu.VMEM((CH, FEAT), jnp.float32),
            pltpu.VMEM((CH,), jnp.int32),
            pltpu.VMEM((NG, FEAT), jnp.float32),
        ],
    )(_seg_max_body)


def _tc_seg_body(sp_ids, x_ref, out_ref):
    i = pl.program_id(0)

    @pl.when(i == 0)
    def _():
        out_ref[...] = jnp.full((NG, FEAT), NEG_INF, jnp.float32)

    base = (TC_FIRST_BLK + i) * TCB
    bend = jnp.minimum(base + TCB, N)
    lo = sp_ids[base]
    hi = sp_ids[bend - 1]
    rows = lax.broadcasted_iota(jnp.int32, (TCB, 1), 0)

    def lower_bound(val):
        def bb(_, c):
            blo, bhi = c
            mid = (blo + bhi) // 2
            v = sp_ids[jnp.minimum(mid, N - 1)]
            active = blo < bhi
            go = jnp.logical_and(active, v < val)
            stay = jnp.logical_and(active, v >= val)
            return (jnp.where(go, mid + 1, blo), jnp.where(stay, mid, bhi))

        return lax.fori_loop(0, TCB.bit_length(), bb, (base, bend))[0]

    wrows = lax.broadcasted_iota(jnp.int32, (TCW, 1), 0)

    def seg_body(s, rs_local):
        re_local = lower_bound(s + 1) - base
        narrow = (re_local - rs_local) <= (TCW - 8)

        @pl.when(narrow)
        def _():
            ws = pl.multiple_of(jnp.minimum(rs_local & ~7, TCB - TCW), 8)
            mask = jnp.logical_and(wrows >= rs_local - ws, wrows < re_local - ws)
            xwin = x_ref[pl.ds(ws, TCW), :]
            m = jnp.max(jnp.where(mask, xwin, NEG_INF), axis=0, keepdims=True)
            out_ref[pl.ds(s, 1), :] = jnp.maximum(out_ref[pl.ds(s, 1), :], m)

        @pl.when(jnp.logical_not(narrow))
        def _():
            mask = jnp.logical_and(rows >= rs_local, rows < re_local)
            m = jnp.max(jnp.where(mask, x_ref[...], NEG_INF), axis=0, keepdims=True)
            out_ref[pl.ds(s, 1), :] = jnp.maximum(out_ref[pl.ds(s, 1), :], m)

        return re_local

    lax.fori_loop(lo, hi + 1, seg_body, jnp.int32(0))


def _tc_seg_max(x, batch_index):
    grid_spec = pltpu.PrefetchScalarGridSpec(
        num_scalar_prefetch=1,
        grid=(TC_BLOCKS,),
        in_specs=[
            pl.BlockSpec((TCB, FEAT), lambda i, sp: (TC_FIRST_BLK + i, 0)),
        ],
        out_specs=pl.BlockSpec((NG, FEAT), lambda i, sp: (0, 0)),
    )
    return pl.pallas_call(
        _tc_seg_body,
        grid_spec=grid_spec,
        out_shape=jax.ShapeDtypeStruct((NG, FEAT), jnp.float32),
    )(batch_index, x)


def _finish_body(part_ref, tc_ref, w_ref, b_ref, out_ref, pooled_ref):
    p = tc_ref[...]
    for i in range(NW):
        p = jnp.maximum(p, part_ref[i])
    pooled_ref[...] = p
    out_ref[...] = (
        jnp.dot(p, w_ref[...], preferred_element_type=jnp.float32) + b_ref[...]
    )


def kernel(x, edge_index, batch_index, Wl0, Wr0, a0, b0, Wls, Wrs, atts, bs,
           Wout, bout):
    partials = _seg_max()(x, batch_index)
    tc_part = _tc_seg_max(x, batch_index)
    out, pooled = pl.pallas_call(
        _finish_body,
        out_shape=(
            jax.ShapeDtypeStruct((NG, NCLS), jnp.float32),
            jax.ShapeDtypeStruct((NG, FEAT), jnp.float32),
        ),
    )(partials, tc_part, Wout, bout.reshape(1, NCLS))
    return (out, pooled)


# speedup vs baseline: 1.1758x; 1.0004x over previous
import functools

import jax
import jax.numpy as jnp
from jax import lax
from jax.experimental import pallas as pl
from jax.experimental.pallas import tpu as pltpu
from jax.experimental.pallas import tpu_sc as plsc

N = 10000
FEAT = 512
NG = 64
NCLS = 10
NEG_INF = float("-inf")

NC = 2
NS = 16
NW = NC * NS
LANE = 16
CH = 64
NCHUNK = 2
TILE_ROWS = CH * NCHUNK
SC_ROWS = NW * TILE_ROWS
NCHW = FEAT // LANE

TCB = 512
TC_FIRST_BLK = 8
TC_BLOCKS = 12
TCW = 256


def _flush(acc, g_cur, regs):
    for c in range(NCHW):
        sl = pl.ds(c * LANE, LANE)
        acc[g_cur, sl] = jnp.maximum(acc[g_cur, sl], regs[c])


def _seg_max_body(x_hbm, ids_hbm, part_hbm, xbuf, ids_v, acc):
    wid = lax.axis_index("c") * NS + lax.axis_index("s")

    def init_g(g, carry):
        for c in range(NCHW):
            acc[g, pl.ds(c * LANE, LANE)] = jnp.full((LANE,), NEG_INF, jnp.float32)
        return carry

    lax.fori_loop(0, NG, init_g, 0)

    base0 = wid * TILE_ROWS
    neg = jnp.full((LANE,), NEG_INF, jnp.float32)
    carry0 = (jnp.int32(0),) + (neg,) * NCHW

    def chunk_body(k, carry):
        base = jnp.minimum(base0 + k * CH, N - CH)
        pltpu.sync_copy(x_hbm.at[pl.ds(base, CH)], xbuf)
        pltpu.sync_copy(ids_hbm.at[pl.ds(base, CH)], ids_v)

        def group_body(rb, carry):
            g_cur = carry[0]
            regs = list(carry[1:])
            idvec = ids_v[pl.ds(rb * LANE, LANE)]
            for j in range(LANE):
                g = idvec[j]
                changed = g != g_cur

                @pl.when(changed)
                def _():
                    _flush(acc, g_cur, regs)

                r = rb * LANE + j
                for c in range(NCHW):
                    row_c = xbuf[r, pl.ds(c * LANE, LANE)]
                    regs[c] = jnp.where(changed, row_c, jnp.maximum(regs[c], row_c))
                g_cur = g
            return (g_cur,) + tuple(regs)

        return lax.fori_loop(0, CH // LANE, group_body, carry)

    carry = lax.fori_loop(0, NCHUNK, chunk_body, carry0)
    _flush(acc, carry[0], list(carry[1:]))
    pltpu.sync_copy(acc, part_hbm.at[wid])


@functools.cache
def _seg_max():
    return functools.partial(
        pl.kernel,
        out_type=jax.ShapeDtypeStruct((NW, NG, FEAT), jnp.float32),
        mesh=plsc.VectorSubcoreMesh(
            core_axis_name="c", subcore_axis_name="s",
            num_cores=NC, num_subcores=NS,
        ),
        scratch_types=[
            pltpu.VMEM((CH, FEAT), jnp.float32),
            pltpu.VMEM((CH,), jnp.int32),
            pltpu.VMEM((NG, FEAT), jnp.float32),
        ],
    )(_seg_max_body)


def _tc_seg_body(sp_ids, x_ref, out_ref):
    i = pl.program_id(0)

    @pl.when(i == 0)
    def _():
        out_ref[...] = jnp.full((NG, FEAT), NEG_INF, jnp.float32)

    base = (TC_FIRST_BLK + i) * TCB
    bend = jnp.minimum(base + TCB, N)
    lo = sp_ids[base]
    hi = sp_ids[bend - 1]
    rows = lax.broadcasted_iota(jnp.int32, (TCB, 1), 0)

    def lower_bound(val):
        def bb(_, c):
            blo, bhi = c
            mid = (blo + bhi) // 2
            v = sp_ids[jnp.minimum(mid, N - 1)]
            active = blo < bhi
            go = jnp.logical_and(active, v < val)
            stay = jnp.logical_and(active, v >= val)
            return (jnp.where(go, mid + 1, blo), jnp.where(stay, mid, bhi))

        return lax.fori_loop(0, TCB.bit_length(), bb, (base, bend))[0]

    wrows = lax.broadcasted_iota(jnp.int32, (TCW, 1), 0)

    def seg_body(s, rs_local):
        re_local = lower_bound(s + 1) - base
        narrow = (re_local - rs_local) <= (TCW - 8)

        @pl.when(narrow)
        def _():
            ws = pl.multiple_of(jnp.minimum(rs_local & ~7, TCB - TCW), 8)
            mask = jnp.logical_and(wrows >= rs_local - ws, wrows < re_local - ws)
            xwin = x_ref[pl.ds(ws, TCW), :]
            m = jnp.max(jnp.where(mask, xwin, NEG_INF), axis=0, keepdims=True)
            out_ref[pl.ds(s, 1), :] = jnp.maximum(out_ref[pl.ds(s, 1), :], m)

        @pl.when(jnp.logical_not(narrow))
        def _():
            mask = jnp.logical_and(rows >= rs_local, rows < re_local)
            m = jnp.max(jnp.where(mask, x_ref[...], NEG_INF), axis=0, keepdims=True)
            out_ref[pl.ds(s, 1), :] = jnp.maximum(out_ref[pl.ds(s, 1), :], m)

        return re_local

    lax.fori_loop(lo, hi + 1, seg_body, jnp.int32(0))


def _tc_seg_max(x, batch_index):
    grid_spec = pltpu.PrefetchScalarGridSpec(
        num_scalar_prefetch=1,
        grid=(TC_BLOCKS,),
        in_specs=[
            pl.BlockSpec((TCB, FEAT), lambda i, sp: (TC_FIRST_BLK + i, 0)),
        ],
        out_specs=pl.BlockSpec((NG, FEAT), lambda i, sp: (0, 0)),
    )
    return pl.pallas_call(
        _tc_seg_body,
        grid_spec=grid_spec,
        out_shape=jax.ShapeDtypeStruct((NG, FEAT), jnp.float32),
    )(batch_index, x)


def _finish_body(part_ref, tc_ref, w_ref, b_ref, out_ref, pooled_ref):
    p = tc_ref[...]
    for i in range(NW):
        p = jnp.maximum(p, part_ref[i])
    pooled_ref[...] = p
    out_ref[...] = (
        jnp.dot(p, w_ref[...], preferred_element_type=jnp.float32) + b_ref[...]
    )


def kernel(x, edge_index, batch_index, Wl0, Wr0, a0, b0, Wls, Wrs, atts, bs,
           Wout, bout):
    partials = _seg_max()(x, batch_index)
    tc_part = _tc_seg_max(x, batch_index)
    out, pooled = pl.pallas_call(
        _finish_body,
        out_shape=(
            jax.ShapeDtypeStruct((NG, NCLS), jnp.float32),
            jax.ShapeDtypeStruct((NG, FEAT), jnp.float32),
        ),
    )(partials, tc_part, Wout, bout.reshape(1, NCLS))
    return (out, pooled)
